# baseline (speedup 1.0000x reference)
import functools

import jax
import jax.numpy as jnp
import numpy as np
from jax import lax
from jax.experimental import pallas as pl
from jax.experimental.pallas import tpu as pltpu

_C0 = (((0,), (0,)), ((), ()))
_MM = (((1,), (0,)), ((), ()))
_HI = lax.Precision.HIGHEST


def _fused_kernel(per_l, per_u, n_l, n_u, emit_lower,
                  xl_ref, xu_ref, bd_ref,
                  out_cross, out_l, out_u,
                  p_acc, rb, mlb_row, mlb_col, vl_row, vl_col,
                  mub_row, mub_col, vu_row, vu_col, q_col, t_scr, rho_scr):
    nb = pl.num_programs(0)
    i = pl.program_id(0)
    j = pl.program_id(1)

    xlj = xl_ref[pl.ds(j * per_l, per_l), :]
    bdb = bd_ref[...]
    ones_l = jnp.ones((per_l, 1), jnp.float32)
    pp = lax.dot_general(bdb.astype(jnp.bfloat16), xlj.astype(jnp.bfloat16),
                         _MM, preferred_element_type=jnp.float32)
    rr = lax.dot_general(bdb, ones_l, _MM, precision=_HI,
                         preferred_element_type=jnp.float32)

    @pl.when(j == 0)
    def _():
        p_acc[...] = pp
        rb[...] = rr

    @pl.when(j > 0)
    def _():
        p_acc[...] += pp
        rb[...] += rr

    @pl.when(i == 0)
    def _lower_stats():
        sq = xlj * xlj
        mlb_row[j] = jnp.sum(xlj, axis=0, keepdims=True)
        mlb_col[j] = lax.dot_general(xlj, ones_l, _C0, precision=_HI,
                                     preferred_element_type=jnp.float32)
        vrow = jnp.sum(sq, axis=0, keepdims=True)
        vcol = lax.dot_general(sq, ones_l, _C0, precision=_HI,
                               preferred_element_type=jnp.float32)

        @pl.when(j == 0)
        def _():
            vl_row[...] = vrow
            vl_col[...] = vcol

        @pl.when(j > 0)
        def _():
            vl_row[...] += vrow
            vl_col[...] += vcol

        if emit_lower:
            xlh = xlj.astype(jnp.bfloat16)
            out_l[j] = lax.dot_general(xlh, xlh, _C0,
                                       preferred_element_type=jnp.float32)

    @pl.when(j == 0)
    def _upper_stats():
        xui = xu_ref[pl.ds(i * per_u, per_u), :]
        ones_u = jnp.ones((per_u, 1), jnp.float32)
        sq = xui * xui
        xuh = xui.astype(jnp.bfloat16)
        out_u[i] = lax.dot_general(xuh, xuh, _C0,
                                   preferred_element_type=jnp.float32)
        mub_row[i] = jnp.sum(xui, axis=0, keepdims=True)
        mub_col[i] = lax.dot_general(xui, ones_u, _C0, precision=_HI,
                                     preferred_element_type=jnp.float32)
        vrow = jnp.sum(sq, axis=0, keepdims=True)
        vcol = lax.dot_general(sq, ones_u, _C0, precision=_HI,
                               preferred_element_type=jnp.float32)

        @pl.when(i == 0)
        def _():
            vu_row[...] = vrow
            vu_col[...] = vcol

        @pl.when(i > 0)
        def _():
            vu_row[...] += vrow
            vu_col[...] += vcol

    @pl.when(j == nb - 1)
    def _collect():
        xui = xu_ref[pl.ds(i * per_u, per_u), :]
        xuh = xui.astype(jnp.bfloat16)
        out_cross[i] = lax.dot_general(xuh, p_acc[...].astype(jnp.bfloat16),
                                       _C0, preferred_element_type=jnp.float32)
        q_col[i] = lax.dot_general(xui, rb[...], _C0, precision=_HI,
                                   preferred_element_type=jnp.float32)
        t_scr[i] = jnp.sum(p_acc[...], axis=0, keepdims=True)
        rho_scr[i] = jnp.sum(rb[...], axis=0, keepdims=True)

    @pl.when((i == nb - 1) & (j == nb - 1))
    def _finalize():
        cl = 1.0 / np.sqrt(n_l - 1)
        cu = 1.0 / np.sqrt(n_u - 1)
        mu_l_row = jnp.sum(mlb_row[...], axis=0) / n_l
        mu_l_col = jnp.sum(mlb_col[...], axis=0) / n_l
        mu_u_row = jnp.sum(mub_row[...], axis=0) / n_u
        mu_u_col = jnp.sum(mub_col[...], axis=0) / n_u
        al_row = cl / (jnp.sqrt((vl_row[...] - n_l * mu_l_row ** 2)
                                / (n_l - 1)) + 1e-6)
        al_col = cl / (jnp.sqrt((vl_col[...] - n_l * mu_l_col ** 2)
                                / (n_l - 1)) + 1e-6)
        au_row = cu / (jnp.sqrt((vu_row[...] - n_u * mu_u_row ** 2)
                                / (n_u - 1)) + 1e-6)
        au_col = cu / (jnp.sqrt((vu_col[...] - n_u * mu_u_col ** 2)
                                / (n_u - 1)) + 1e-6)
        for b_ in range(nb):
            if emit_lower:
                out_l[b_] = al_col * al_row * (
                    out_l[b_] - mlb_col[b_] * mu_l_row
                    - mu_l_col * mlb_row[b_] + per_l * mu_l_col * mu_l_row)
            out_u[b_] = au_col * au_row * (
                out_u[b_] - mub_col[b_] * mu_u_row
                - mu_u_col * mub_row[b_] + per_u * mu_u_col * mu_u_row)
            out_cross[b_] = au_col * al_row * (
                out_cross[b_] - q_col[b_] * mu_l_row
                - mu_u_col * t_scr[b_] + rho_scr[b_] * mu_u_col * mu_l_row)


def _cross_call(Xl, Xu, Bdry, b, emit_lower):
    per_l = Xl.shape[0] // b
    per_u = Xu.shape[0] // b
    n_l, n_u = Xl.shape[0], Xu.shape[0]
    d = Xl.shape[1]
    out_sh = jax.ShapeDtypeStruct((b, d, d), jnp.float32)
    corr_spec = pl.BlockSpec((b, d, d), lambda i, j: (0, 0, 0))
    f32 = jnp.float32
    return pl.pallas_call(
        functools.partial(_fused_kernel, per_l, per_u, n_l, n_u, emit_lower),
        grid=(b, b),
        in_specs=[
            pl.BlockSpec((n_l, d), lambda i, j: (0, 0)),
            pl.BlockSpec((n_u, d), lambda i, j: (0, 0)),
            pl.BlockSpec((per_u, per_l), lambda i, j: (i, j)),
        ],
        out_specs=[corr_spec, corr_spec, corr_spec],
        out_shape=[out_sh, out_sh, out_sh],
        scratch_shapes=[
            pltpu.VMEM((per_u, d), f32),
            pltpu.VMEM((per_u, 1), f32),
            pltpu.VMEM((b, 1, d), f32),
            pltpu.VMEM((b, d, 1), f32),
            pltpu.VMEM((1, d), f32),
            pltpu.VMEM((d, 1), f32),
            pltpu.VMEM((b, 1, d), f32),
            pltpu.VMEM((b, d, 1), f32),
            pltpu.VMEM((1, d), f32),
            pltpu.VMEM((d, 1), f32),
            pltpu.VMEM((b, d, 1), f32),
            pltpu.VMEM((b, 1, d), f32),
            pltpu.VMEM((b, 1, 1), f32),
        ],
        compiler_params=pltpu.CompilerParams(
            dimension_semantics=("arbitrary", "arbitrary")),
    )(Xl, Xu, Bdry)


def kernel(X0, X1, X2, D2B1TD1inv, B2TD2inv, num_nodes, num_edges,
           num_triangles):
    b = len(num_nodes)
    X01corr, X0corr, X1corr = _cross_call(X0, X1, D2B1TD1inv, b, True)
    X12corr, _, X2corr = _cross_call(X1, X2, B2TD2inv, b, False)
    return (X0corr, X1corr, X2corr, X01corr, X12corr)


# speedup vs baseline: 2.2533x; 2.2533x over previous
import functools

import jax
import jax.numpy as jnp
import numpy as np
from jax import lax
from jax.experimental import pallas as pl
from jax.experimental.pallas import tpu as pltpu

_C0 = (((0,), (0,)), ((), ()))
_MM = (((1,), (0,)), ((), ()))


def _colstats(x, n):
    mu = jnp.sum(x, axis=0, keepdims=True) / n
    v = jnp.sum(x * x, axis=0, keepdims=True)
    var = (v - n * mu * mu) / (n - 1)
    alpha = (1.0 / np.sqrt(n - 1)) / (jnp.sqrt(var) + 1e-6)
    return mu, alpha


def _fused_kernel(per_l, per_u, n_l, n_u, emit_lower,
                  xl_ref, xu_ref, bd_ref,
                  out_cross, out_l, out_u,
                  p_acc, mu_l, al_l, mu_u, al_u):
    nb = pl.num_programs(0)
    i = pl.program_id(0)
    j = pl.program_id(1)

    @pl.when((i == 0) & (j == 0))
    def _stats():
        mu, al = _colstats(xl_ref[...], n_l)
        mu_l[...] = mu
        al_l[...] = al
        mu, al = _colstats(xu_ref[...], n_u)
        mu_u[...] = mu
        al_u[...] = al

    xlj = xl_ref[pl.ds(j * per_l, per_l), :]
    ylj = ((xlj - mu_l[...]) * al_l[...]).astype(jnp.bfloat16)
    pp = lax.dot_general(bd_ref[...].astype(jnp.bfloat16), ylj, _MM,
                         preferred_element_type=jnp.float32)

    @pl.when(j == 0)
    def _():
        p_acc[...] = pp

    @pl.when(j > 0)
    def _():
        p_acc[...] += pp

    if emit_lower:
        @pl.when(i == 0)
        def _lower_self():
            out_l[j] = lax.dot_general(ylj, ylj, _C0,
                                       preferred_element_type=jnp.float32)

    @pl.when(j == 0)
    def _upper_self():
        xui = xu_ref[pl.ds(i * per_u, per_u), :]
        yui = ((xui - mu_u[...]) * al_u[...]).astype(jnp.bfloat16)
        out_u[i] = lax.dot_general(yui, yui, _C0,
                                   preferred_element_type=jnp.float32)

    @pl.when(j == nb - 1)
    def _cross():
        xui = xu_ref[pl.ds(i * per_u, per_u), :]
        yui = ((xui - mu_u[...]) * al_u[...]).astype(jnp.bfloat16)
        out_cross[i] = lax.dot_general(yui, p_acc[...].astype(jnp.bfloat16),
                                       _C0, preferred_element_type=jnp.float32)


def _cross_call(Xl, Xu, Bdry, b, emit_lower):
    per_l = Xl.shape[0] // b
    per_u = Xu.shape[0] // b
    n_l, n_u = Xl.shape[0], Xu.shape[0]
    d = Xl.shape[1]
    out_sh = jax.ShapeDtypeStruct((b, d, d), jnp.float32)
    corr_spec = pl.BlockSpec((b, d, d), lambda i, j: (0, 0, 0))
    f32 = jnp.float32
    return pl.pallas_call(
        functools.partial(_fused_kernel, per_l, per_u, n_l, n_u, emit_lower),
        grid=(b, b),
        in_specs=[
            pl.BlockSpec((n_l, d), lambda i, j: (0, 0)),
            pl.BlockSpec((n_u, d), lambda i, j: (0, 0)),
            pl.BlockSpec((per_u, per_l), lambda i, j: (i, j)),
        ],
        out_specs=[corr_spec, corr_spec, corr_spec],
        out_shape=[out_sh, out_sh, out_sh],
        scratch_shapes=[
            pltpu.VMEM((per_u, d), f32),
            pltpu.VMEM((1, d), f32),
            pltpu.VMEM((1, d), f32),
            pltpu.VMEM((1, d), f32),
            pltpu.VMEM((1, d), f32),
        ],
        compiler_params=pltpu.CompilerParams(
            dimension_semantics=("arbitrary", "arbitrary")),
    )(Xl, Xu, Bdry)


def kernel(X0, X1, X2, D2B1TD1inv, B2TD2inv, num_nodes, num_edges,
           num_triangles):
    b = len(num_nodes)
    X01corr, X0corr, X1corr = _cross_call(X0, X1, D2B1TD1inv, b, True)
    X12corr, _, X2corr = _cross_call(X1, X2, B2TD2inv, b, False)
    return (X0corr, X1corr, X2corr, X01corr, X12corr)


# speedup vs baseline: 3.5187x; 1.5616x over previous
import functools

import jax
import jax.numpy as jnp
import numpy as np
from jax import lax
from jax.experimental import pallas as pl
from jax.experimental.pallas import tpu as pltpu

_C0 = (((0,), (0,)), ((), ()))
_MM = (((1,), (0,)), ((), ()))


def _colstats(x, n):
    mu = jnp.sum(x, axis=0, keepdims=True) / n
    v = jnp.sum(x * x, axis=0, keepdims=True)
    var = (v - n * mu * mu) / (n - 1)
    alpha = (1.0 / np.sqrt(n - 1)) / (jnp.sqrt(var) + 1e-6)
    return mu, alpha


def _fused_kernel(per_l, per_u, n_l, n_u, emit_lower,
                  xl_ref, xu_ref, bd_ref,
                  out_cross, out_l, out_u,
                  p_acc, mu_l, al_l, mu_u, al_u):
    nk = pl.num_programs(1)
    i = pl.program_id(0)
    k = pl.program_id(1)
    half = n_l // nk

    @pl.when((i == 0) & (k == 0))
    def _stats():
        mu, al = _colstats(xl_ref[...], n_l)
        mu_l[...] = mu
        al_l[...] = al
        mu, al = _colstats(xu_ref[...], n_u)
        mu_u[...] = mu
        al_u[...] = al

    xlk = xl_ref[pl.ds(k * half, half), :]
    ylk = ((xlk - mu_l[...]) * al_l[...]).astype(jnp.bfloat16)
    pp = lax.dot_general(bd_ref[...].astype(jnp.bfloat16), ylk, _MM,
                         preferred_element_type=jnp.float32)

    @pl.when(k == 0)
    def _():
        p_acc[...] = pp

    @pl.when(k > 0)
    def _():
        p_acc[...] += pp

    if emit_lower:
        @pl.when(i == 0)
        def _lower_self():
            nbl = half // per_l
            for b2 in range(nbl):
                yb = ylk[b2 * per_l:(b2 + 1) * per_l, :]
                out_l[nbl * k + b2] = lax.dot_general(
                    yb, yb, _C0, preferred_element_type=jnp.float32)

    @pl.when(k == 0)
    def _upper_self():
        xui = xu_ref[pl.ds(i * per_u, per_u), :]
        yui = ((xui - mu_u[...]) * al_u[...]).astype(jnp.bfloat16)
        out_u[i] = lax.dot_general(yui, yui, _C0,
                                   preferred_element_type=jnp.float32)

    @pl.when(k == nk - 1)
    def _cross():
        xui = xu_ref[pl.ds(i * per_u, per_u), :]
        yui = ((xui - mu_u[...]) * al_u[...]).astype(jnp.bfloat16)
        out_cross[i] = lax.dot_general(yui, p_acc[...].astype(jnp.bfloat16),
                                       _C0, preferred_element_type=jnp.float32)


def _cross_call(Xl, Xu, Bdry, b, emit_lower, nk=2):
    per_l = Xl.shape[0] // b
    per_u = Xu.shape[0] // b
    n_l, n_u = Xl.shape[0], Xu.shape[0]
    d = Xl.shape[1]
    out_sh = jax.ShapeDtypeStruct((b, d, d), jnp.float32)
    corr_spec = pl.BlockSpec((b, d, d), lambda i, j: (0, 0, 0))
    f32 = jnp.float32
    return pl.pallas_call(
        functools.partial(_fused_kernel, per_l, per_u, n_l, n_u, emit_lower),
        grid=(b, nk),
        in_specs=[
            pl.BlockSpec((n_l, d), lambda i, j: (0, 0)),
            pl.BlockSpec((n_u, d), lambda i, j: (0, 0)),
            pl.BlockSpec((per_u, n_l // nk), lambda i, j: (i, j)),
        ],
        out_specs=[corr_spec, corr_spec, corr_spec],
        out_shape=[out_sh, out_sh, out_sh],
        scratch_shapes=[
            pltpu.VMEM((per_u, d), f32),
            pltpu.VMEM((1, d), f32),
            pltpu.VMEM((1, d), f32),
            pltpu.VMEM((1, d), f32),
            pltpu.VMEM((1, d), f32),
        ],
        compiler_params=pltpu.CompilerParams(
            dimension_semantics=("arbitrary", "arbitrary")),
    )(Xl, Xu, Bdry)


def kernel(X0, X1, X2, D2B1TD1inv, B2TD2inv, num_nodes, num_edges,
           num_triangles):
    b = len(num_nodes)
    X01corr, X0corr, X1corr = _cross_call(X0, X1, D2B1TD1inv, b, True)
    X12corr, _, X2corr = _cross_call(X1, X2, B2TD2inv, b, False)
    return (X0corr, X1corr, X2corr, X01corr, X12corr)


# speedup vs baseline: 3.7264x; 1.0590x over previous
import functools

import jax
import jax.numpy as jnp
import numpy as np
from jax import lax
from jax.experimental import pallas as pl
from jax.experimental.pallas import tpu as pltpu

_C0 = (((0,), (0,)), ((), ()))
_MM = (((1,), (0,)), ((), ()))


def _colstats(x, n):
    mu = jnp.sum(x, axis=0, keepdims=True) / n
    v = jnp.sum(x * x, axis=0, keepdims=True)
    var = (v - n * mu * mu) / (n - 1)
    alpha = (1.0 / np.sqrt(n - 1)) / (jnp.sqrt(var) + 1e-6)
    return mu, alpha


def _fused_kernel(per_l, per_u, n_l, n_u, emit_lower,
                  xl_ref, xu_ref, bd_ref,
                  out_cross, out_l, out_u,
                  p_acc, mu_l, al_l, mu_u, al_u):
    nk = pl.num_programs(1)
    i = pl.program_id(0)
    k = pl.program_id(1)
    half = n_l // nk

    @pl.when((i == 0) & (k == 0))
    def _stats():
        mu, al = _colstats(xl_ref[...], n_l)
        mu_l[...] = mu
        al_l[...] = al
        mu, al = _colstats(xu_ref[...], n_u)
        mu_u[...] = mu
        al_u[...] = al

    xlk = xl_ref[pl.ds(k * half, half), :]
    ylk = ((xlk - mu_l[...]) * al_l[...]).astype(jnp.bfloat16)
    pp = lax.dot_general(bd_ref[...].astype(jnp.bfloat16), ylk, _MM,
                         preferred_element_type=jnp.float32)

    @pl.when(k == 0)
    def _():
        p_acc[...] = pp

    @pl.when(k > 0)
    def _():
        p_acc[...] += pp

    if emit_lower:
        @pl.when(i == 0)
        def _lower_self():
            nbl = half // per_l
            for b2 in range(nbl):
                yb = ylk[b2 * per_l:(b2 + 1) * per_l, :]
                out_l[nbl * k + b2] = lax.dot_general(
                    yb, yb, _C0, preferred_element_type=jnp.float32)

    @pl.when(k == 0)
    def _upper_self():
        xui = xu_ref[pl.ds(i * per_u, per_u), :]
        yui = ((xui - mu_u[...]) * al_u[...]).astype(jnp.bfloat16)
        out_u[i] = lax.dot_general(yui, yui, _C0,
                                   preferred_element_type=jnp.float32)

    @pl.when(k == nk - 1)
    def _cross():
        xui = xu_ref[pl.ds(i * per_u, per_u), :]
        yui = ((xui - mu_u[...]) * al_u[...]).astype(jnp.bfloat16)
        out_cross[i] = lax.dot_general(yui, p_acc[...].astype(jnp.bfloat16),
                                       _C0, preferred_element_type=jnp.float32)


def _cross_call(Xl, Xu, Bdry, b, emit_lower, nk=2):
    per_l = Xl.shape[0] // b
    per_u = Xu.shape[0] // b
    n_l, n_u = Xl.shape[0], Xu.shape[0]
    d = Xl.shape[1]
    out_sh = jax.ShapeDtypeStruct((b, d, d), jnp.float32)
    corr_spec = pl.BlockSpec((b, d, d), lambda i, j: (0, 0, 0))
    f32 = jnp.float32
    return pl.pallas_call(
        functools.partial(_fused_kernel, per_l, per_u, n_l, n_u, emit_lower),
        grid=(b, nk),
        in_specs=[
            pl.BlockSpec((n_l, d), lambda i, j: (0, 0)),
            pl.BlockSpec((n_u, d), lambda i, j: (0, 0)),
            pl.BlockSpec((per_u, n_l // nk), lambda i, j: (i, j)),
        ],
        out_specs=[corr_spec, corr_spec, corr_spec],
        out_shape=[out_sh, out_sh, out_sh],
        scratch_shapes=[
            pltpu.VMEM((per_u, d), f32),
            pltpu.VMEM((1, d), f32),
            pltpu.VMEM((1, d), f32),
            pltpu.VMEM((1, d), f32),
            pltpu.VMEM((1, d), f32),
        ],
        compiler_params=pltpu.CompilerParams(
            dimension_semantics=("arbitrary", "arbitrary")),
    )(Xl, Xu, Bdry)


def kernel(X0, X1, X2, D2B1TD1inv, B2TD2inv, num_nodes, num_edges,
           num_triangles):
    b = len(num_nodes)
    X01corr, X0corr, X1corr = _cross_call(X0, X1, D2B1TD1inv, b, True, nk=1)
    X12corr, _, X2corr = _cross_call(X1, X2, B2TD2inv, b, False, nk=1)
    return (X0corr, X1corr, X2corr, X01corr, X12corr)
